# baseline (speedup 1.0000x reference)
import functools

import jax
import jax.numpy as jnp
from jax import lax
from jax.experimental import pallas as pl
from jax.experimental.pallas import tpu as pltpu
from jax.experimental.pallas import tpu_sc as plsc

_N = 10000
_E = 320000
_D = 128
_NP = 10240
_NC = 2
_NS = 16
_NW = _NC * _NS
_L = 16
_B = 128
_NB = 80
_EPB = _NB * _B
_EPT = _E // _NW
_SL = _NP // _NS
_NBUF = 2

_mesh = plsc.VectorSubcoreMesh(core_axis_name="c", subcore_axis_name="s")


_EPC = _E // _NS
_SLH = _SL // 2


@functools.partial(
    pl.kernel,
    out_type=jax.ShapeDtypeStruct((_NP, _D), jnp.float32),
    mesh=_mesh,
    compiler_params=pltpu.CompilerParams(needs_layout_passes=False),
    scratch_types=[
        pltpu.VMEM((_EPC,), jnp.int32),
        pltpu.VMEM((_NP,), jnp.float32),
        pltpu.VMEM((_NS, _SL), jnp.float32),
        pltpu.VMEM((_B, _D), jnp.float32),
        pltpu.VMEM_SHARED((_NS, _NP), jnp.float32),
    ],
)
def _deg_kernel(dst_hbm, out_hbm, dst_v, deg_v, comb_v, degm_v, stage_sh):
    cid = lax.axis_index("c")
    sid = lax.axis_index("s")
    pltpu.sync_copy(dst_hbm.at[pl.ds(sid * _EPC, _EPC)], dst_v)

    zeros = jnp.zeros((_L,), jnp.float32)

    def _zero(i, carry):
        deg_v[pl.ds(i * _L, _L)] = zeros
        return carry

    lax.fori_loop(0, _NP // _L, _zero, 0)

    ones = jnp.ones((_L,), jnp.float32)

    def _count(i, carry):
        idx = dst_v[pl.ds(i * _L, _L)]
        plsc.addupdate_scatter(deg_v, [idx], ones)
        return carry

    lax.fori_loop(0, _EPC // _L, _count, 0)

    pltpu.sync_copy(deg_v, stage_sh.at[sid])
    plsc.subcore_barrier()

    @pl.when(sid < _NS // 2)
    def _finalize():
        base = cid * (_NP // 2) + sid * _SL
        pltpu.sync_copy(stage_sh.at[:, pl.ds(base, _SL)], comb_v)

        def _sum(j, carry):
            acc = comb_v[0, pl.ds(j * _L, _L)]
            for r in range(1, _NS):
                acc = acc + comb_v[r, pl.ds(j * _L, _L)]
            deg_v[pl.ds(j * _L, _L)] = acc
            return carry

        lax.fori_loop(0, _SL // _L, _sum, 0)

        for t in range(_SL // _B):
            def _bcast(j, carry):
                vec = deg_v[pl.ds(t * _B + j * _L, _L)]
                for l in range(_L):
                    row = jnp.full((_L,), 1.0, jnp.float32) * vec[l]
                    k = j * _L + l
                    for kk in range(_D // _L):
                        degm_v[k, pl.ds(kk * _L, _L)] = row
                return carry

            lax.fori_loop(0, _B // _L, _bcast, 0)
            pltpu.sync_copy(degm_v, out_hbm.at[pl.ds(base + t * _B, _B)])


@functools.partial(
    pl.kernel,
    out_type=jax.ShapeDtypeStruct((2 * _NP, _D), jnp.float32),
    mesh=_mesh,
    compiler_params=pltpu.CompilerParams(needs_layout_passes=False),
    scratch_types=[
        pltpu.VMEM((_NB // 2, _B), jnp.int32),
        pltpu.VMEM((_NB // 2, _B), jnp.int32),
        pltpu.VMEM((_NBUF, _B, _D), jnp.float32),
        pltpu.VMEM_SHARED((_NP, _D), jnp.float32),
        pltpu.SemaphoreType.DMA,
        pltpu.SemaphoreType.DMA,
    ],
)
def _scatter_kernel(p_hbm, srcp_hbm, dstp_hbm, out_hbm,
                    src_v, dst_v, rows_v, agg_sh, sem0, sem1):
    cid = lax.axis_index("c")
    sid = lax.axis_index("s")
    wid = cid * _NS + sid
    nbh = _NB // 2

    zeros = jnp.zeros((_L,), jnp.float32)

    def _zrow(i, carry):
        for k in range(_D // _L):
            rows_v[0, i, pl.ds(k * _L, _L)] = zeros
        return carry

    lax.fori_loop(0, _B, _zrow, 0)
    for t in range(_SL // _B):
        pltpu.sync_copy(rows_v.at[0], agg_sh.at[pl.ds(sid * _SL + t * _B, _B)])
    plsc.subcore_barrier()

    sems = (sem0, sem1)

    for h in range(2):
        pltpu.sync_copy(srcp_hbm.at[wid, pl.ds(h * nbh, nbh)], src_v)
        pltpu.sync_copy(dstp_hbm.at[wid, pl.ds(h * nbh, nbh)], dst_v)

        for b in range(_NBUF):
            pltpu.async_copy(p_hbm.at[src_v.at[b]], rows_v.at[b], sems[b])

        def _step(g, carry):
            for b in range(_NBUF):
                j = g * _NBUF + b
                pltpu.make_async_copy(p_hbm.at[src_v.at[j]], rows_v.at[b],
                                      sems[b]).wait()
                pltpu.sync_copy(rows_v.at[b], agg_sh.at[dst_v.at[j]], add=True)

                @pl.when(j + _NBUF < nbh)
                def _():
                    pltpu.async_copy(p_hbm.at[src_v.at[j + _NBUF]],
                                     rows_v.at[b], sems[b])

            return carry

        lax.fori_loop(0, nbh // _NBUF, _step, 0)
    plsc.subcore_barrier()
    pltpu.sync_copy(agg_sh.at[pl.ds(sid * _SL, _SL)],
                    out_hbm.at[pl.ds(cid * _NP + sid * _SL, _SL)])


_BM = 640
_NBLK = _NP // _BM


def _mm_body(feat_ref, w_ref, d_ref, o_ref):
    deg = d_ref[...] + 1.0
    dinv = lax.rsqrt(jnp.maximum(deg, 1.0))
    o_ref[...] = jnp.dot(feat_ref[...] * dinv, w_ref[...],
                         preferred_element_type=jnp.float32)


_mm = pl.pallas_call(
    _mm_body,
    grid=(_NBLK,),
    in_specs=[
        pl.BlockSpec((_BM, _D), lambda i: (i, 0)),
        pl.BlockSpec((_D, _D), lambda i: (0, 0)),
        pl.BlockSpec((_BM, _D), lambda i: (i, 0)),
    ],
    out_specs=pl.BlockSpec((_BM, _D), lambda i: (i, 0)),
    out_shape=jax.ShapeDtypeStruct((_NP, _D), jnp.float32),
)


def _fin_body(s0_ref, s1_ref, p_ref, d_ref, b_ref, o_ref):
    deg = d_ref[...] + 1.0
    dinv = lax.rsqrt(jnp.maximum(deg, 1.0))
    o_ref[...] = dinv * (s0_ref[...] + s1_ref[...] + p_ref[...]) \
        + b_ref[...][None, :]


_fin = pl.pallas_call(
    _fin_body,
    grid=(_NBLK,),
    in_specs=[
        pl.BlockSpec((_BM, _D), lambda i: (i, 0)),
        pl.BlockSpec((_BM, _D), lambda i: (i + _NBLK, 0)),
        pl.BlockSpec((_BM, _D), lambda i: (i, 0)),
        pl.BlockSpec((_BM, _D), lambda i: (i, 0)),
        pl.BlockSpec((_D,), lambda i: (0,)),
    ],
    out_specs=pl.BlockSpec((_BM, _D), lambda i: (i, 0)),
    out_shape=jax.ShapeDtypeStruct((_N, _D), jnp.float32),
)


def kernel(feat, edge_index, weight, bias):
    n, d_in = feat.shape
    src = edge_index[0]
    dst = edge_index[1]
    pad_e = _NW * _EPB - dst.shape[0]
    fill = _N + (jnp.arange(pad_e, dtype=jnp.int32) % (_NP - _N))
    srcp = jnp.concatenate([src, fill]).reshape(_NW, _NB, _B)
    dstp = jnp.concatenate([dst, fill]).reshape(_NW, _NB, _B)

    degm = _deg_kernel(dst)
    p = _mm(feat, weight, degm)
    s = _scatter_kernel(p, srcp, dstp)
    return _fin(s, s, p, degm, bias)


# speedup vs baseline: 1.0150x; 1.0150x over previous
import functools

import jax
import jax.numpy as jnp
from jax import lax
from jax.experimental import pallas as pl
from jax.experimental.pallas import tpu as pltpu
from jax.experimental.pallas import tpu_sc as plsc

_N = 10000
_E = 320000
_D = 128
_NP = 10240
_NC = 2
_NS = 16
_NW = _NC * _NS
_L = 16
_B = 128
_NB = 80
_EPB = _NB * _B
_EPT = _E // _NW
_SL = _NP // _NS
_NBUF = 2

_mesh = plsc.VectorSubcoreMesh(core_axis_name="c", subcore_axis_name="s")


@functools.partial(
    pl.kernel,
    out_type=jax.ShapeDtypeStruct((2 * _NP, _D), jnp.float32),
    mesh=_mesh,
    compiler_params=pltpu.CompilerParams(needs_layout_passes=False),
    scratch_types=[
        pltpu.VMEM((_EPT,), jnp.int32),
        pltpu.VMEM((_NP,), jnp.float32),
        pltpu.VMEM((_NS, _SL), jnp.float32),
        pltpu.VMEM((_B, _D), jnp.float32),
        pltpu.VMEM_SHARED((_NS, _NP), jnp.float32),
    ],
)
def _deg_kernel(dst_hbm, out_hbm, dst_v, deg_v, comb_v, degm_v, stage_sh):
    cid = lax.axis_index("c")
    sid = lax.axis_index("s")
    wid = cid * _NS + sid
    pltpu.sync_copy(dst_hbm.at[pl.ds(wid * _EPT, _EPT)], dst_v)

    zeros = jnp.zeros((_L,), jnp.float32)

    def _zero(i, carry):
        deg_v[pl.ds(i * _L, _L)] = zeros
        return carry

    lax.fori_loop(0, _NP // _L, _zero, 0)

    ones = jnp.ones((_L,), jnp.float32)

    def _count(i, carry):
        idx = dst_v[pl.ds(i * _L, _L)]
        plsc.addupdate_scatter(deg_v, [idx], ones)
        return carry

    lax.fori_loop(0, _EPT // _L, _count, 0)

    pltpu.sync_copy(deg_v, stage_sh.at[sid])
    plsc.subcore_barrier()
    pltpu.sync_copy(stage_sh.at[:, pl.ds(sid * _SL, _SL)], comb_v)

    def _sum(j, carry):
        acc = comb_v[0, pl.ds(j * _L, _L)]
        for r in range(1, _NS):
            acc = acc + comb_v[r, pl.ds(j * _L, _L)]
        deg_v[pl.ds(j * _L, _L)] = acc
        return carry

    lax.fori_loop(0, _SL // _L, _sum, 0)

    for t in range(_SL // _B):
        def _bcast(j, carry):
            vec = deg_v[pl.ds(t * _B + j * _L, _L)]
            for l in range(_L):
                row = jnp.full((_L,), 1.0, jnp.float32) * vec[l]
                k = j * _L + l
                for kk in range(_D // _L):
                    degm_v[k, pl.ds(kk * _L, _L)] = row
            return carry

        lax.fori_loop(0, _B // _L, _bcast, 0)
        pltpu.sync_copy(
            degm_v,
            out_hbm.at[pl.ds(cid * _NP + sid * _SL + t * _B, _B)])


@functools.partial(
    pl.kernel,
    out_type=jax.ShapeDtypeStruct((2 * _NP, _D), jnp.float32),
    mesh=_mesh,
    compiler_params=pltpu.CompilerParams(needs_layout_passes=False),
    scratch_types=[
        pltpu.VMEM((_NB // 2, _B), jnp.int32),
        pltpu.VMEM((_NB // 2, _B), jnp.int32),
        pltpu.VMEM((_NBUF, _B, _D), jnp.float32),
        pltpu.VMEM_SHARED((_NP, _D), jnp.float32),
        pltpu.SemaphoreType.DMA,
        pltpu.SemaphoreType.DMA,
    ],
)
def _scatter_kernel(p_hbm, srcp_hbm, dstp_hbm, out_hbm,
                    src_v, dst_v, rows_v, agg_sh, sem0, sem1):
    cid = lax.axis_index("c")
    sid = lax.axis_index("s")
    wid = cid * _NS + sid
    nbh = _NB // 2

    zeros = jnp.zeros((_L,), jnp.float32)

    def _zrow(i, carry):
        for k in range(_D // _L):
            rows_v[0, i, pl.ds(k * _L, _L)] = zeros
        return carry

    lax.fori_loop(0, _B, _zrow, 0)
    for t in range(_SL // _B):
        pltpu.sync_copy(rows_v.at[0], agg_sh.at[pl.ds(sid * _SL + t * _B, _B)])
    plsc.subcore_barrier()

    sems = (sem0, sem1)

    for h in range(2):
        pltpu.sync_copy(srcp_hbm.at[wid, pl.ds(h * nbh, nbh)], src_v)
        pltpu.sync_copy(dstp_hbm.at[wid, pl.ds(h * nbh, nbh)], dst_v)

        for b in range(_NBUF):
            pltpu.async_copy(p_hbm.at[src_v.at[b]], rows_v.at[b], sems[b])

        def _step(g, carry):
            for b in range(_NBUF):
                j = g * _NBUF + b
                pltpu.make_async_copy(p_hbm.at[src_v.at[j]], rows_v.at[b],
                                      sems[b]).wait()
                pltpu.sync_copy(rows_v.at[b], agg_sh.at[dst_v.at[j]], add=True)

                @pl.when(j + _NBUF < nbh)
                def _():
                    pltpu.async_copy(p_hbm.at[src_v.at[j + _NBUF]],
                                     rows_v.at[b], sems[b])

            return carry

        lax.fori_loop(0, nbh // _NBUF, _step, 0)
    plsc.subcore_barrier()
    pltpu.sync_copy(agg_sh.at[pl.ds(sid * _SL, _SL)],
                    out_hbm.at[pl.ds(cid * _NP + sid * _SL, _SL)])


_BM = 640
_NBLK = _NP // _BM


def _mm_body(feat_ref, w_ref, d0_ref, d1_ref, o_ref):
    deg = d0_ref[...] + d1_ref[...] + 1.0
    dinv = lax.rsqrt(jnp.maximum(deg, 1.0))
    o_ref[...] = jnp.dot(feat_ref[...] * dinv, w_ref[...],
                         preferred_element_type=jnp.float32)


_mm = pl.pallas_call(
    _mm_body,
    grid=(_NBLK,),
    in_specs=[
        pl.BlockSpec((_BM, _D), lambda i: (i, 0)),
        pl.BlockSpec((_D, _D), lambda i: (0, 0)),
        pl.BlockSpec((_BM, _D), lambda i: (i, 0)),
        pl.BlockSpec((_BM, _D), lambda i: (i + _NBLK, 0)),
    ],
    out_specs=pl.BlockSpec((_BM, _D), lambda i: (i, 0)),
    out_shape=jax.ShapeDtypeStruct((_NP, _D), jnp.float32),
)


def _fin_body(s0_ref, s1_ref, p_ref, d0_ref, d1_ref, b_ref, o_ref):
    deg = d0_ref[...] + d1_ref[...] + 1.0
    dinv = lax.rsqrt(jnp.maximum(deg, 1.0))
    o_ref[...] = dinv * (s0_ref[...] + s1_ref[...] + p_ref[...]) \
        + b_ref[...][None, :]


_fin = pl.pallas_call(
    _fin_body,
    grid=(_NBLK,),
    in_specs=[
        pl.BlockSpec((_BM, _D), lambda i: (i, 0)),
        pl.BlockSpec((_BM, _D), lambda i: (i + _NBLK, 0)),
        pl.BlockSpec((_BM, _D), lambda i: (i, 0)),
        pl.BlockSpec((_BM, _D), lambda i: (i, 0)),
        pl.BlockSpec((_BM, _D), lambda i: (i + _NBLK, 0)),
        pl.BlockSpec((_D,), lambda i: (0,)),
    ],
    out_specs=pl.BlockSpec((_BM, _D), lambda i: (i, 0)),
    out_shape=jax.ShapeDtypeStruct((_N, _D), jnp.float32),
)


def kernel(feat, edge_index, weight, bias):
    n, d_in = feat.shape
    src = edge_index[0]
    dst = edge_index[1]
    pad_e = _NW * _EPB - dst.shape[0]
    fill = _N + (jnp.arange(pad_e, dtype=jnp.int32) % (_NP - _N))
    srcp = jnp.concatenate([src, fill]).reshape(_NW, _NB, _B)
    dstp = jnp.concatenate([dst, fill]).reshape(_NW, _NB, _B)

    degm = _deg_kernel(dst)
    p = _mm(feat, weight, degm, degm)
    s = _scatter_kernel(p, srcp, dstp)
    return _fin(s, s, p, degm, degm, bias)
